# baseline (speedup 1.0000x reference)
import functools

import jax
import jax.numpy as jnp
from jax import lax
from jax.experimental import pallas as pl
from jax.experimental.pallas import tpu as pltpu
from jax.experimental.pallas import tpu_sc as plsc

_NUM_CORES = 2
_NUM_SUBCORES = 16
_NW = _NUM_CORES * _NUM_SUBCORES
_CHUNK = 128
_NBUF = 4


def _make_repack_kernel(vocab, hidden):
  n_full = vocab // 128
  tail = vocab - n_full * 128
  base_blocks = n_full // _NW
  extra = n_full - base_blocks * _NW
  mesh = plsc.VectorSubcoreMesh(core_axis_name="c", subcore_axis_name="s")

  @functools.partial(
      pl.kernel,
      out_type=jax.ShapeDtypeStruct((vocab // 2, 128), jnp.float32),
      mesh=mesh,
      scratch_types=[
          pltpu.VMEM((2, hidden, 128), jnp.float32),
          pltpu.VMEM((hidden, 128), jnp.float32),
          pltpu.SemaphoreType.DMA((2,)),
      ],
      compiler_params=pltpu.CompilerParams(needs_layout_passes=False),
  )
  def kt(tab_t_hbm, tail_hbm, out_hbm, inblk, tblk, sems):
    wid = lax.axis_index("s") * _NUM_CORES + lax.axis_index("c")
    nblocks = jnp.where(wid < extra, base_blocks + 1, base_blocks)
    iota16 = lax.iota(jnp.int32, 16)

    pltpu.async_copy(
        tab_t_hbm.at[:, pl.ds(wid * 128, 128)], inblk.at[0], sems.at[0])

    def do_transpose(buf, width):
      @pl.loop(0, width // 2)
      def rows(p):
        for c in range(8):
          rix = iota16 + (16 * (c % 4))
          cix = jnp.full((16,), 2 * p + (1 if c >= 4 else 0), jnp.int32)
          tblk[p, pl.ds(16 * c, 16)] = plsc.load_gather(buf, [rix, cix])

    def body(j, carry):
      t = wid + j * _NW
      b = lax.rem(j, 2)
      pltpu.make_async_copy(
          tab_t_hbm.at[:, pl.ds(t * 128, 128)], inblk.at[b], sems.at[b]).wait()

      @pl.when(j + 1 < nblocks)
      def _():
        nt = t + _NW
        pltpu.async_copy(
            tab_t_hbm.at[:, pl.ds(nt * 128, 128)], inblk.at[1 - b],
            sems.at[1 - b])

      do_transpose(inblk.at[b], 128)
      pltpu.sync_copy(tblk, out_hbm.at[pl.ds(t * 64, 64)])
      return carry

    lax.fori_loop(0, nblocks, body, 0)

    @pl.when(wid == _NW - 1)
    def _():
      pltpu.sync_copy(tail_hbm, tblk.at[pl.ds(0, tail // 2)])
      pltpu.sync_copy(
          tblk.at[pl.ds(0, tail // 2)],
          out_hbm.at[pl.ds(n_full * 64, tail // 2)])

  return kt


def _make_gather_kernel(n_chunks, chunk, seq, hidden, total):
  per_w = n_chunks * chunk
  mesh = plsc.VectorSubcoreMesh(core_axis_name="c", subcore_axis_name="s")

  @functools.partial(
      pl.kernel,
      out_type=jax.ShapeDtypeStruct((total, 128), jnp.float32),
      mesh=mesh,
      scratch_types=[
          pltpu.VMEM((n_chunks, chunk), jnp.int32),
          pltpu.VMEM((2 * seq, hidden), jnp.float32),
          pltpu.VMEM((_NBUF, chunk, hidden), jnp.float32),
          pltpu.SemaphoreType.DMA((_NBUF,)),
      ],
      compiler_params=pltpu.CompilerParams(use_tc_tiling_on_sc=False),
  )
  def k(idx_hbm, table_hbm, pos2_hbm, out_hbm, idx_v, pos_v, rows_v, sems):
    wid = lax.axis_index("s") * _NUM_CORES + lax.axis_index("c")
    pltpu.sync_copy(idx_hbm.at[wid], idx_v)
    pltpu.sync_copy(pos2_hbm, pos_v)

    for b in range(_NBUF):
      pltpu.async_copy(table_hbm.at[idx_v.at[b]], rows_v.at[b], sems.at[b])

    def outer(t, carry):
      for b in range(_NBUF):
        jj = t * _NBUF + b
        pltpu.make_async_copy(
            table_hbm.at[idx_v.at[jj]], rows_v.at[b], sems.at[b]).wait()
        p = lax.rem(jj * chunk, seq)

        @plsc.parallel_loop(0, chunk, unroll=8)
        def add_row(i):
          for c in range(hidden // 16):
            sl = pl.ds(c * 16, 16)
            plsc.addupdate(rows_v.at[b, i, sl], pos_v[p + i, sl])

        pltpu.sync_copy(
            rows_v.at[b],
            out_hbm.at[pl.ds(wid * per_w + jj * chunk, chunk),
                       pl.ds(0, hidden)])

        nj = jj + _NBUF

        @pl.when(nj < n_chunks)
        def _():
          pltpu.async_copy(
              table_hbm.at[idx_v.at[nj]], rows_v.at[b], sems.at[b])

      return carry

    lax.fori_loop(0, n_chunks // _NBUF, outer, 0)

  return k


def kernel(token_ids, token_embed, position_embed):
  batch, seq = token_ids.shape
  vocab, hidden = token_embed.shape
  total = batch * seq
  n_chunks = total // (_NW * _CHUNK)

  idx3 = token_ids.reshape(_NW, n_chunks, _CHUNK).astype(jnp.int32)
  pos = position_embed[0, :seq].astype(jnp.float32)
  pos2 = jnp.concatenate([pos, pos], axis=0)

  kt = _make_repack_kernel(vocab, hidden)
  n_full = vocab // 128
  tail2 = token_embed[n_full * 128:].reshape(-1, 128)
  packed = kt(token_embed.T, tail2)
  table_lin = packed.reshape(vocab, hidden)

  k = _make_gather_kernel(n_chunks, _CHUNK, seq, hidden, total)
  out = k(idx3, table_lin, pos2)
  return out[:, :hidden].reshape(batch, seq, hidden)


# speedup vs baseline: 2.8431x; 2.8431x over previous
import functools

import jax
import jax.numpy as jnp
from jax import lax
from jax.experimental import pallas as pl
from jax.experimental.pallas import tpu as pltpu
from jax.experimental.pallas import tpu_sc as plsc

_NUM_CORES = 2
_NUM_SUBCORES = 16
_NW = _NUM_CORES * _NUM_SUBCORES
_CHUNK = 128
_NBUF = 4


def _make_repack_kernel(vocab, hidden):
  n_full = vocab // 128
  tail = vocab - n_full * 128
  base_blocks = n_full // _NW
  extra = n_full - base_blocks * _NW
  mesh = plsc.VectorSubcoreMesh(core_axis_name="c", subcore_axis_name="s")

  @functools.partial(
      pl.kernel,
      out_type=jax.ShapeDtypeStruct((vocab // 2, 128), jnp.float32),
      mesh=mesh,
      scratch_types=[
          pltpu.VMEM((2, hidden, 128), jnp.float32),
          pltpu.VMEM((2, hidden, 128), jnp.float32),
          pltpu.SemaphoreType.DMA((2,)),
          pltpu.SemaphoreType.DMA((2,)),
      ],
      compiler_params=pltpu.CompilerParams(needs_layout_passes=False),
  )
  def kt(tab_t_hbm, tail_hbm, out_hbm, inblk, tblk, isems, osems):
    wid = lax.axis_index("s") * _NUM_CORES + lax.axis_index("c")
    nblocks = jnp.where(wid < extra, base_blocks + 1, base_blocks)
    iota16 = lax.iota(jnp.int32, 16)

    pltpu.async_copy(
        tab_t_hbm.at[:, pl.ds(wid * 128, 128)], inblk.at[0], isems.at[0])

    def do_transpose(buf, tbuf):
      @functools.partial(plsc.parallel_loop, 0, 64, unroll=4)
      def rows(p):
        base = jnp.full((16,), 2 * p, jnp.int32)
        base1 = base + 1
        for c in range(8):
          rix = iota16 + (16 * (c % 4))
          cix = base if c < 4 else base1
          tbuf[p, pl.ds(16 * c, 16)] = plsc.load_gather(buf, [rix, cix])

    def body(j2, carry):
      for b in range(2):
        j = j2 * 2 + b

        @pl.when(j < nblocks)
        def _():
          t = wid + j * _NW
          pltpu.make_async_copy(
              tab_t_hbm.at[:, pl.ds(t * 128, 128)], inblk.at[b],
              isems.at[b]).wait()

          @pl.when(j + 1 < nblocks)
          def _():
            pltpu.async_copy(
                tab_t_hbm.at[:, pl.ds((t + _NW) * 128, 128)],
                inblk.at[1 - b], isems.at[1 - b])

          @pl.when(j >= 2)
          def _():
            pltpu.make_async_copy(
                tblk.at[b], out_hbm.at[pl.ds((t - 2 * _NW) * 64, 64)],
                osems.at[b]).wait()

          do_transpose(inblk.at[b], tblk.at[b])
          pltpu.async_copy(
              tblk.at[b], out_hbm.at[pl.ds(t * 64, 64)], osems.at[b])

      return carry

    lax.fori_loop(0, (base_blocks + 2) // 2, body, 0)

    for b in range(2):
      jl = ((nblocks - 1 - b) // 2) * 2 + b

      @pl.when(jl >= 0)
      def _():
        tl = wid + jl * _NW
        pltpu.make_async_copy(
            tblk.at[b], out_hbm.at[pl.ds(tl * 64, 64)], osems.at[b]).wait()

    @pl.when(wid == _NW - 1)
    def _():
      pltpu.sync_copy(tail_hbm, inblk.at[0, pl.ds(0, tail // 2)])
      pltpu.sync_copy(
          inblk.at[0, pl.ds(0, tail // 2)],
          out_hbm.at[pl.ds(n_full * 64, tail // 2)])

  return kt


def _make_gather_kernel(n_chunks, chunk, seq, hidden, total):
  per_w = n_chunks * chunk
  mesh = plsc.VectorSubcoreMesh(core_axis_name="c", subcore_axis_name="s")

  @functools.partial(
      pl.kernel,
      out_type=jax.ShapeDtypeStruct((total, 128), jnp.float32),
      mesh=mesh,
      scratch_types=[
          pltpu.VMEM((n_chunks, chunk), jnp.int32),
          pltpu.VMEM((2 * seq, hidden), jnp.float32),
          pltpu.VMEM((_NBUF, chunk, hidden), jnp.float32),
          pltpu.SemaphoreType.DMA((_NBUF,)),
      ],
      compiler_params=pltpu.CompilerParams(use_tc_tiling_on_sc=False),
  )
  def k(idx_hbm, table_hbm, pos2_hbm, out_hbm, idx_v, pos_v, rows_v, sems):
    wid = lax.axis_index("s") * _NUM_CORES + lax.axis_index("c")
    pltpu.sync_copy(idx_hbm.at[wid], idx_v)
    pltpu.sync_copy(pos2_hbm, pos_v)

    for b in range(_NBUF):
      pltpu.async_copy(table_hbm.at[idx_v.at[b]], rows_v.at[b], sems.at[b])

    def outer(t, carry):
      for b in range(_NBUF):
        jj = t * _NBUF + b
        pltpu.make_async_copy(
            table_hbm.at[idx_v.at[jj]], rows_v.at[b], sems.at[b]).wait()
        p = lax.rem(jj * chunk, seq)

        @plsc.parallel_loop(0, chunk, unroll=8)
        def add_row(i):
          for c in range(hidden // 16):
            sl = pl.ds(c * 16, 16)
            plsc.addupdate(rows_v.at[b, i, sl], pos_v[p + i, sl])

        pltpu.sync_copy(
            rows_v.at[b],
            out_hbm.at[pl.ds(wid * per_w + jj * chunk, chunk),
                       pl.ds(0, hidden)])

        nj = jj + _NBUF

        @pl.when(nj < n_chunks)
        def _():
          pltpu.async_copy(
              table_hbm.at[idx_v.at[nj]], rows_v.at[b], sems.at[b])

      return carry

    lax.fori_loop(0, n_chunks // _NBUF, outer, 0)

  return k


def kernel(token_ids, token_embed, position_embed):
  batch, seq = token_ids.shape
  vocab, hidden = token_embed.shape
  total = batch * seq
  n_chunks = total // (_NW * _CHUNK)

  idx3 = token_ids.reshape(_NW, n_chunks, _CHUNK).astype(jnp.int32)
  pos = position_embed[0, :seq].astype(jnp.float32)
  pos2 = jnp.concatenate([pos, pos], axis=0)

  kt = _make_repack_kernel(vocab, hidden)
  n_full = vocab // 128
  tail2 = token_embed[n_full * 128:].reshape(-1, 128)
  packed = kt(token_embed.T, tail2)
  table_lin = packed.reshape(vocab, hidden)

  k = _make_gather_kernel(n_chunks, _CHUNK, seq, hidden, total)
  out = k(idx3, table_lin, pos2)
  return out[:, :hidden].reshape(batch, seq, hidden)
